# baseline (speedup 1.0000x reference)
import functools

import jax
import jax.numpy as jnp
from jax import lax
from jax.experimental import pallas as pl
from jax.experimental.pallas import tpu as pltpu
from jax.experimental.pallas import tpu_sc as plsc

_N = 10000
_E = 320000
_D = 128
_W1 = 144

_NC = 2
_NS = 16
_NW = _NC * _NS
_CH = 128
_SLAB = 4
_NSLAB = 20
_CPW = _SLAB * _NSLAB
_EPAD = _NW * _CPW * _CH
_RPT = 640
_NACC = _NS * _RPT

_BN = 1000


def _make_sc_segsum(width):
    mesh = plsc.VectorSubcoreMesh(core_axis_name="c", subcore_axis_name="s")

    @functools.partial(
        pl.kernel,
        out_type=jax.ShapeDtypeStruct((_NC, _NACC, width), jnp.float32),
        mesh=mesh,
        scratch_types=[
            pltpu.VMEM((_SLAB, _CH), jnp.int32),
            pltpu.VMEM((_SLAB, _CH), jnp.int32),
            pltpu.VMEM((_CH, width), jnp.float32),
            pltpu.VMEM_SHARED((_NACC, width), jnp.float32),
            pltpu.SemaphoreType.DMA,
        ],
        compiler_params=pltpu.CompilerParams(use_tc_tiling_on_sc=False),
    )
    def sc_kernel(table, srcs, dsts, zeros, out,
                  src_v, dst_v, rows, acc, sem):
        c = lax.axis_index("c")
        s = lax.axis_index("s")
        w = s * _NC + c
        pltpu.sync_copy(zeros, acc.at[pl.ds(s * _RPT, _RPT)])
        plsc.subcore_barrier()

        def slab_step(sl, carry):
            pltpu.sync_copy(srcs.at[w, pl.ds(sl * _SLAB, _SLAB)], src_v)
            pltpu.sync_copy(dsts.at[w, pl.ds(sl * _SLAB, _SLAB)], dst_v)
            for j in range(_SLAB):
                pltpu.async_copy(table.at[src_v.at[j]], rows, sem).wait()
                pltpu.sync_copy(rows, acc.at[dst_v.at[j]], add=True)
            return carry

        lax.fori_loop(0, _NSLAB, slab_step, 0)
        plsc.subcore_barrier()
        pltpu.sync_copy(acc.at[pl.ds(s * _RPT, _RPT)],
                        out.at[c, pl.ds(s * _RPT, _RPT)])

    return sc_kernel


_sc_segsum_aug = _make_sc_segsum(_W1)
_sc_segsum_plain = _make_sc_segsum(_D)


def _tc_layer1(P, x, W_l, W_r, b):
    def body(p_ref, x_ref, wl_ref, wr_ref, b_ref, h_ref, rdeg_ref):
        p = p_ref[0] + p_ref[1]
        agg = p[:, :_D]
        deg = p[:, _D:_D + 1]
        rdeg = 1.0 / jnp.maximum(deg, 1.0)
        hv = (jnp.dot(agg * rdeg, wl_ref[:], preferred_element_type=jnp.float32)
              + jnp.dot(x_ref[:], wr_ref[:], preferred_element_type=jnp.float32)
              + b_ref[:])
        h_ref[:] = jnp.maximum(hv, 0.0)
        rdeg_ref[:] = rdeg

    return pl.pallas_call(
        body,
        grid=(_N // _BN,),
        in_specs=[
            pl.BlockSpec((_NC, _BN, _W1), lambda i: (0, i, 0)),
            pl.BlockSpec((_BN, _D), lambda i: (i, 0)),
            pl.BlockSpec((_D, _D), lambda i: (0, 0)),
            pl.BlockSpec((_D, _D), lambda i: (0, 0)),
            pl.BlockSpec((1, _D), lambda i: (0, 0)),
        ],
        out_specs=[
            pl.BlockSpec((_BN, _D), lambda i: (i, 0)),
            pl.BlockSpec((_BN, 1), lambda i: (i, 0)),
        ],
        out_shape=[
            jax.ShapeDtypeStruct((_N, _D), jnp.float32),
            jax.ShapeDtypeStruct((_N, 1), jnp.float32),
        ],
    )(P, x, W_l, W_r, b)


def _tc_layer2(P, h, rdeg, W_l, W_r, b):
    def body(p_ref, h_ref, rdeg_ref, wl_ref, wr_ref, b_ref, o_ref):
        agg = p_ref[0] + p_ref[1]
        o_ref[:] = (jnp.dot(agg * rdeg_ref[:], wl_ref[:],
                            preferred_element_type=jnp.float32)
                    + jnp.dot(h_ref[:], wr_ref[:],
                              preferred_element_type=jnp.float32)
                    + b_ref[:])

    return pl.pallas_call(
        body,
        grid=(_N // _BN,),
        in_specs=[
            pl.BlockSpec((_NC, _BN, _D), lambda i: (0, i, 0)),
            pl.BlockSpec((_BN, _D), lambda i: (i, 0)),
            pl.BlockSpec((_BN, 1), lambda i: (i, 0)),
            pl.BlockSpec((_D, _D), lambda i: (0, 0)),
            pl.BlockSpec((_D, _D), lambda i: (0, 0)),
            pl.BlockSpec((1, _D), lambda i: (0, 0)),
        ],
        out_specs=pl.BlockSpec((_BN, _D), lambda i: (i, 0)),
        out_shape=jax.ShapeDtypeStruct((_N, _D), jnp.float32),
    )(P, h, rdeg, W_l, W_r, b)


def kernel(x, edge_index, W1_l, W1_r, b1, W2_l, W2_r, b2):
    src = edge_index[0]
    dst = edge_index[1]
    pad = _EPAD - _E
    src_p = jnp.concatenate(
        [src, jnp.zeros((pad,), jnp.int32)]).reshape(
            _CPW, _NW, _CH).transpose(1, 0, 2)
    pad_dst = _N + (jnp.arange(pad, dtype=jnp.int32) % (_NACC - _N))
    dst_p = jnp.concatenate([dst, pad_dst]).reshape(
        _CPW, _NW, _CH).transpose(1, 0, 2)
    xaug = jnp.concatenate(
        [x, jnp.ones((_N, 1), jnp.float32), jnp.zeros((_N, _W1 - _D - 1),
                                                      jnp.float32)], axis=1)
    z_aug = jnp.zeros((_RPT, _W1), jnp.float32)
    z_plain = jnp.zeros((_RPT, _D), jnp.float32)

    P1 = _sc_segsum_aug(xaug, src_p, dst_p, z_aug)
    h, rdeg = _tc_layer1(P1, x, W1_l, W1_r, b1.reshape(1, _D))
    P2 = _sc_segsum_plain(h, src_p, dst_p, z_plain)
    return _tc_layer2(P2, h, rdeg, W2_l, W2_r, b2.reshape(1, _D))


# speedup vs baseline: 1.0329x; 1.0329x over previous
import functools

import jax
import jax.numpy as jnp
from jax import lax
from jax.experimental import pallas as pl
from jax.experimental.pallas import tpu as pltpu
from jax.experimental.pallas import tpu_sc as plsc

_N = 10000
_E = 320000
_D = 128
_W1 = 144

_NC = 2
_NS = 16
_NW = _NC * _NS
_CH = 128
_SLAB = 4
_NSLAB = 20
_CPW = _SLAB * _NSLAB
_EPAD = _NW * _CPW * _CH
_RPT = 638
_NACC = _NS * _RPT

_BN = 1000


def _make_sc_segsum(width):
    mesh = plsc.VectorSubcoreMesh(core_axis_name="c", subcore_axis_name="s")

    @functools.partial(
        pl.kernel,
        out_type=jax.ShapeDtypeStruct((_NC, _NACC, width), jnp.float32),
        mesh=mesh,
        scratch_types=[
            pltpu.VMEM((_CPW, _CH), jnp.int32),
            pltpu.VMEM((_CPW, _CH), jnp.int32),
            pltpu.VMEM((_CH, width), jnp.float32),
            pltpu.VMEM_SHARED((_NACC, width), jnp.float32),
            pltpu.SemaphoreType.DMA,
        ],
        compiler_params=pltpu.CompilerParams(use_tc_tiling_on_sc=False),
    )
    def sc_kernel(table, srcs, dsts, zeros, out,
                  src_v, dst_v, rows, acc, sem):
        c = lax.axis_index("c")
        s = lax.axis_index("s")
        w = s * _NC + c
        pltpu.sync_copy(srcs.at[w], src_v)
        pltpu.sync_copy(dsts.at[w], dst_v)
        pltpu.sync_copy(zeros, acc.at[pl.ds(s * _RPT, _RPT)])
        plsc.subcore_barrier()

        def step(g, carry):
            pltpu.async_copy(table.at[src_v.at[g]], rows, sem).wait()
            pltpu.sync_copy(rows, acc.at[dst_v.at[g]], add=True)
            return carry

        lax.fori_loop(0, _CPW, step, 0)
        plsc.subcore_barrier()
        pltpu.sync_copy(acc.at[pl.ds(s * _RPT, _RPT)],
                        out.at[c, pl.ds(s * _RPT, _RPT)])

    return sc_kernel


_sc_segsum_aug = _make_sc_segsum(_W1)
_sc_segsum_plain = _make_sc_segsum(_D)


def _tc_layer1(P, x, W_l, W_r, b):
    def body(p_ref, x_ref, wl_ref, wr_ref, b_ref, h_ref, rdeg_ref):
        p = p_ref[0] + p_ref[1]
        agg = p[:, :_D]
        deg = p[:, _D:_D + 1]
        rdeg = 1.0 / jnp.maximum(deg, 1.0)
        hv = (jnp.dot(agg * rdeg, wl_ref[:], preferred_element_type=jnp.float32)
              + jnp.dot(x_ref[:], wr_ref[:], preferred_element_type=jnp.float32)
              + b_ref[:])
        h_ref[:] = jnp.maximum(hv, 0.0)
        rdeg_ref[:] = rdeg

    return pl.pallas_call(
        body,
        grid=(_N // _BN,),
        in_specs=[
            pl.BlockSpec((_NC, _BN, _W1), lambda i: (0, i, 0)),
            pl.BlockSpec((_BN, _D), lambda i: (i, 0)),
            pl.BlockSpec((_D, _D), lambda i: (0, 0)),
            pl.BlockSpec((_D, _D), lambda i: (0, 0)),
            pl.BlockSpec((1, _D), lambda i: (0, 0)),
        ],
        out_specs=[
            pl.BlockSpec((_BN, _D), lambda i: (i, 0)),
            pl.BlockSpec((_BN, 1), lambda i: (i, 0)),
        ],
        out_shape=[
            jax.ShapeDtypeStruct((_N, _D), jnp.float32),
            jax.ShapeDtypeStruct((_N, 1), jnp.float32),
        ],
    )(P, x, W_l, W_r, b)


def _tc_layer2(P, h, rdeg, W_l, W_r, b):
    def body(p_ref, h_ref, rdeg_ref, wl_ref, wr_ref, b_ref, o_ref):
        agg = p_ref[0] + p_ref[1]
        o_ref[:] = (jnp.dot(agg * rdeg_ref[:], wl_ref[:],
                            preferred_element_type=jnp.float32)
                    + jnp.dot(h_ref[:], wr_ref[:],
                              preferred_element_type=jnp.float32)
                    + b_ref[:])

    return pl.pallas_call(
        body,
        grid=(_N // _BN,),
        in_specs=[
            pl.BlockSpec((_NC, _BN, _D), lambda i: (0, i, 0)),
            pl.BlockSpec((_BN, _D), lambda i: (i, 0)),
            pl.BlockSpec((_BN, 1), lambda i: (i, 0)),
            pl.BlockSpec((_D, _D), lambda i: (0, 0)),
            pl.BlockSpec((_D, _D), lambda i: (0, 0)),
            pl.BlockSpec((1, _D), lambda i: (0, 0)),
        ],
        out_specs=pl.BlockSpec((_BN, _D), lambda i: (i, 0)),
        out_shape=jax.ShapeDtypeStruct((_N, _D), jnp.float32),
    )(P, h, rdeg, W_l, W_r, b)


def kernel(x, edge_index, W1_l, W1_r, b1, W2_l, W2_r, b2):
    src = edge_index[0]
    dst = edge_index[1]
    pad = _EPAD - _E
    src_p = jnp.concatenate(
        [src, jnp.zeros((pad,), jnp.int32)]).reshape(
            _CPW, _NW, _CH).transpose(1, 0, 2)
    pad_dst = _N + (jnp.arange(pad, dtype=jnp.int32) % (_NACC - _N))
    dst_p = jnp.concatenate([dst, pad_dst]).reshape(
        _CPW, _NW, _CH).transpose(1, 0, 2)
    xaug = jnp.concatenate(
        [x, jnp.ones((_N, 1), jnp.float32), jnp.zeros((_N, _W1 - _D - 1),
                                                      jnp.float32)], axis=1)
    z_aug = jnp.zeros((_RPT, _W1), jnp.float32)
    z_plain = jnp.zeros((_RPT, _D), jnp.float32)

    P1 = _sc_segsum_aug(xaug, src_p, dst_p, z_aug)
    h, rdeg = _tc_layer1(P1, x, W1_l, W1_r, b1.reshape(1, _D))
    P2 = _sc_segsum_plain(h, src_p, dst_p, z_plain)
    return _tc_layer2(P2, h, rdeg, W2_l, W2_r, b2.reshape(1, _D))


# speedup vs baseline: 1.2119x; 1.1733x over previous
import functools

import jax
import jax.numpy as jnp
from jax import lax
from jax.experimental import pallas as pl
from jax.experimental.pallas import tpu as pltpu
from jax.experimental.pallas import tpu_sc as plsc

_N = 10000
_E = 320000
_D = 128
_W1 = 144

_NC = 2
_NS = 16
_NW = _NC * _NS
_CH = 64
_CPW = 158
_EPAD = _NW * _CPW * _CH
_RPT = 640
_NACC = _NS * _RPT

_BN = 1000


def _make_sc_segsum(width):
    mesh = plsc.VectorSubcoreMesh(core_axis_name="c", subcore_axis_name="s")

    @functools.partial(
        pl.kernel,
        out_type=jax.ShapeDtypeStruct((_NC, _NACC, width), jnp.float32),
        mesh=mesh,
        scratch_types=[
            pltpu.VMEM((_CPW, _CH), jnp.int32),
            pltpu.VMEM((_CPW, _CH), jnp.int32),
            pltpu.VMEM((_CH, width), jnp.float32),
            pltpu.VMEM((_CH, width), jnp.float32),
            pltpu.VMEM_SHARED((_NACC, width), jnp.float32),
            pltpu.SemaphoreType.DMA,
            pltpu.SemaphoreType.DMA,
        ],
        compiler_params=pltpu.CompilerParams(use_tc_tiling_on_sc=False),
    )
    def sc_kernel(table, srcs, dsts, zeros, out,
                  src_v, dst_v, rows0, rows1, acc, sem0, sem1):
        c = lax.axis_index("c")
        s = lax.axis_index("s")
        w = s * _NC + c
        pltpu.sync_copy(srcs.at[w], src_v)
        pltpu.sync_copy(dsts.at[w], dst_v)
        pltpu.sync_copy(zeros, acc.at[pl.ds(s * _RPT, _RPT)])
        plsc.subcore_barrier()

        pltpu.async_copy(table.at[src_v.at[0]], rows0, sem0)

        def step(i, carry):
            g0 = 2 * i
            pltpu.make_async_copy(table.at[src_v.at[g0]], rows0, sem0).wait()
            pltpu.async_copy(table.at[src_v.at[g0 + 1]], rows1, sem1)
            pltpu.sync_copy(rows0, acc.at[dst_v.at[g0]], add=True)
            g2 = jnp.minimum(g0 + 2, _CPW - 1)
            pltpu.make_async_copy(table.at[src_v.at[g0 + 1]], rows1, sem1).wait()
            pltpu.async_copy(table.at[src_v.at[g2]], rows0, sem0)
            pltpu.sync_copy(rows1, acc.at[dst_v.at[g0 + 1]], add=True)
            return carry

        lax.fori_loop(0, _CPW // 2, step, 0)
        pltpu.make_async_copy(table.at[src_v.at[_CPW - 1]], rows0, sem0).wait()
        plsc.subcore_barrier()
        pltpu.sync_copy(acc.at[pl.ds(s * _RPT, _RPT)],
                        out.at[c, pl.ds(s * _RPT, _RPT)])

    return sc_kernel


_sc_segsum_aug = _make_sc_segsum(_W1)
_sc_segsum_plain = _make_sc_segsum(_D)


def _tc_layer1(P, x, W_l, W_r, b):
    def body(p_ref, x_ref, wl_ref, wr_ref, b_ref, h_ref, rdeg_ref):
        p = p_ref[0] + p_ref[1]
        agg = p[:, :_D]
        deg = p[:, _D:_D + 1]
        rdeg = 1.0 / jnp.maximum(deg, 1.0)
        hv = (jnp.dot(agg * rdeg, wl_ref[:], preferred_element_type=jnp.float32)
              + jnp.dot(x_ref[:], wr_ref[:], preferred_element_type=jnp.float32)
              + b_ref[:])
        h_ref[:] = jnp.maximum(hv, 0.0)
        rdeg_ref[:] = rdeg

    return pl.pallas_call(
        body,
        grid=(_N // _BN,),
        in_specs=[
            pl.BlockSpec((_NC, _BN, _W1), lambda i: (0, i, 0)),
            pl.BlockSpec((_BN, _D), lambda i: (i, 0)),
            pl.BlockSpec((_D, _D), lambda i: (0, 0)),
            pl.BlockSpec((_D, _D), lambda i: (0, 0)),
            pl.BlockSpec((1, _D), lambda i: (0, 0)),
        ],
        out_specs=[
            pl.BlockSpec((_BN, _D), lambda i: (i, 0)),
            pl.BlockSpec((_BN, 1), lambda i: (i, 0)),
        ],
        out_shape=[
            jax.ShapeDtypeStruct((_N, _D), jnp.float32),
            jax.ShapeDtypeStruct((_N, 1), jnp.float32),
        ],
    )(P, x, W_l, W_r, b)


def _tc_layer2(P, h, rdeg, W_l, W_r, b):
    def body(p_ref, h_ref, rdeg_ref, wl_ref, wr_ref, b_ref, o_ref):
        agg = p_ref[0] + p_ref[1]
        o_ref[:] = (jnp.dot(agg * rdeg_ref[:], wl_ref[:],
                            preferred_element_type=jnp.float32)
                    + jnp.dot(h_ref[:], wr_ref[:],
                              preferred_element_type=jnp.float32)
                    + b_ref[:])

    return pl.pallas_call(
        body,
        grid=(_N // _BN,),
        in_specs=[
            pl.BlockSpec((_NC, _BN, _D), lambda i: (0, i, 0)),
            pl.BlockSpec((_BN, _D), lambda i: (i, 0)),
            pl.BlockSpec((_BN, 1), lambda i: (i, 0)),
            pl.BlockSpec((_D, _D), lambda i: (0, 0)),
            pl.BlockSpec((_D, _D), lambda i: (0, 0)),
            pl.BlockSpec((1, _D), lambda i: (0, 0)),
        ],
        out_specs=pl.BlockSpec((_BN, _D), lambda i: (i, 0)),
        out_shape=jax.ShapeDtypeStruct((_N, _D), jnp.float32),
    )(P, h, rdeg, W_l, W_r, b)


def kernel(x, edge_index, W1_l, W1_r, b1, W2_l, W2_r, b2):
    src = edge_index[0]
    dst = edge_index[1]
    pad = _EPAD - _E
    src_p = jnp.concatenate(
        [src, jnp.zeros((pad,), jnp.int32)]).reshape(
            _CPW, _NW, _CH).transpose(1, 0, 2)
    pad_dst = _N + (jnp.arange(pad, dtype=jnp.int32) % (_NACC - _N))
    dst_p = jnp.concatenate([dst, pad_dst]).reshape(
        _CPW, _NW, _CH).transpose(1, 0, 2)
    xaug = jnp.concatenate(
        [x, jnp.ones((_N, 1), jnp.float32), jnp.zeros((_N, _W1 - _D - 1),
                                                      jnp.float32)], axis=1)
    z_aug = jnp.zeros((_RPT, _W1), jnp.float32)
    z_plain = jnp.zeros((_RPT, _D), jnp.float32)

    P1 = _sc_segsum_aug(xaug, src_p, dst_p, z_aug)
    h, rdeg = _tc_layer1(P1, x, W1_l, W1_r, b1.reshape(1, _D))
    P2 = _sc_segsum_plain(h, src_p, dst_p, z_plain)
    return _tc_layer2(P2, h, rdeg, W2_l, W2_r, b2.reshape(1, _D))


# speedup vs baseline: 1.4593x; 1.2041x over previous
import functools

import jax
import jax.numpy as jnp
from jax import lax
from jax.experimental import pallas as pl
from jax.experimental.pallas import tpu as pltpu
from jax.experimental.pallas import tpu_sc as plsc

_N = 10000
_E = 320000
_D = 128
_W1 = 144

_NC = 2
_NS = 16
_NW = _NC * _NS
_CH = 128
_CPW = 79
_EPAD = _NW * _CPW * _CH
_RPT = 640
_NACC = _NS * _RPT

_BN = 1000


def _make_sc_segsum(width):
    mesh = plsc.VectorSubcoreMesh(core_axis_name="c", subcore_axis_name="s")

    @functools.partial(
        pl.kernel,
        out_type=jax.ShapeDtypeStruct((_NC, _NACC, width), jnp.float32),
        mesh=mesh,
        scratch_types=[
            pltpu.VMEM((_CPW, _CH), jnp.int32),
            pltpu.VMEM((_CPW, _CH), jnp.int32),
            pltpu.VMEM((_CH, width), jnp.float32),
            pltpu.VMEM_SHARED((_NACC, width), jnp.float32),
            pltpu.SemaphoreType.DMA,
        ],
        compiler_params=pltpu.CompilerParams(use_tc_tiling_on_sc=False),
    )
    def sc_kernel(table, srcs, dsts, zeros, out,
                  src_v, dst_v, rows, acc, sem):
        c = lax.axis_index("c")
        s = lax.axis_index("s")
        w = s * _NC + c
        pltpu.sync_copy(srcs.at[w], src_v)
        pltpu.sync_copy(dsts.at[w], dst_v)
        pltpu.sync_copy(zeros, acc.at[pl.ds(s * _RPT, _RPT)])
        plsc.subcore_barrier()

        def step(g, carry):
            pltpu.async_copy(table.at[src_v.at[g]], rows, sem).wait()
            pltpu.sync_copy(rows, acc.at[dst_v.at[g]], add=True)
            return carry

        lax.fori_loop(0, _CPW, step, 0)
        plsc.subcore_barrier()
        pltpu.sync_copy(acc.at[pl.ds(s * _RPT, _RPT)],
                        out.at[c, pl.ds(s * _RPT, _RPT)])

    return sc_kernel


_sc_segsum_aug = _make_sc_segsum(_W1)
_sc_segsum_plain = _make_sc_segsum(_D)


def _tc_layer1(P, x, W_l, W_r, b):
    def body(p_ref, x_ref, wl_ref, wr_ref, b_ref, h_ref, rdeg_ref):
        p = p_ref[0] + p_ref[1]
        agg = p[:, :_D]
        deg = p[:, _D:_D + 1]
        rdeg = 1.0 / jnp.maximum(deg, 1.0)
        hv = (jnp.dot(agg * rdeg, wl_ref[:], preferred_element_type=jnp.float32)
              + jnp.dot(x_ref[:], wr_ref[:], preferred_element_type=jnp.float32)
              + b_ref[:])
        h_ref[:] = jnp.maximum(hv, 0.0)
        rdeg_ref[:] = rdeg

    return pl.pallas_call(
        body,
        grid=(_N // _BN,),
        in_specs=[
            pl.BlockSpec((_NC, _BN, _W1), lambda i: (0, i, 0)),
            pl.BlockSpec((_BN, _D), lambda i: (i, 0)),
            pl.BlockSpec((_D, _D), lambda i: (0, 0)),
            pl.BlockSpec((_D, _D), lambda i: (0, 0)),
            pl.BlockSpec((1, _D), lambda i: (0, 0)),
        ],
        out_specs=[
            pl.BlockSpec((_BN, _D), lambda i: (i, 0)),
            pl.BlockSpec((_BN, 1), lambda i: (i, 0)),
        ],
        out_shape=[
            jax.ShapeDtypeStruct((_N, _D), jnp.float32),
            jax.ShapeDtypeStruct((_N, 1), jnp.float32),
        ],
    )(P, x, W_l, W_r, b)


def _tc_layer2(P, h, rdeg, W_l, W_r, b):
    def body(p_ref, h_ref, rdeg_ref, wl_ref, wr_ref, b_ref, o_ref):
        agg = p_ref[0] + p_ref[1]
        o_ref[:] = (jnp.dot(agg * rdeg_ref[:], wl_ref[:],
                            preferred_element_type=jnp.float32)
                    + jnp.dot(h_ref[:], wr_ref[:],
                              preferred_element_type=jnp.float32)
                    + b_ref[:])

    return pl.pallas_call(
        body,
        grid=(_N // _BN,),
        in_specs=[
            pl.BlockSpec((_NC, _BN, _D), lambda i: (0, i, 0)),
            pl.BlockSpec((_BN, _D), lambda i: (i, 0)),
            pl.BlockSpec((_BN, 1), lambda i: (i, 0)),
            pl.BlockSpec((_D, _D), lambda i: (0, 0)),
            pl.BlockSpec((_D, _D), lambda i: (0, 0)),
            pl.BlockSpec((1, _D), lambda i: (0, 0)),
        ],
        out_specs=pl.BlockSpec((_BN, _D), lambda i: (i, 0)),
        out_shape=jax.ShapeDtypeStruct((_N, _D), jnp.float32),
    )(P, h, rdeg, W_l, W_r, b)


def kernel(x, edge_index, W1_l, W1_r, b1, W2_l, W2_r, b2):
    src = edge_index[0]
    dst = edge_index[1]
    pad = _EPAD - _E
    src_p = jnp.concatenate(
        [src, jnp.zeros((pad,), jnp.int32)]).reshape(
            _CPW, _NW, _CH).transpose(1, 0, 2)
    pad_dst = _N + (jnp.arange(pad, dtype=jnp.int32) % (_NACC - _N))
    dst_p = jnp.concatenate([dst, pad_dst]).reshape(
        _CPW, _NW, _CH).transpose(1, 0, 2)
    xaug = jnp.concatenate(
        [x, jnp.ones((_N, 1), jnp.float32), jnp.zeros((_N, _W1 - _D - 1),
                                                      jnp.float32)], axis=1)
    z_aug = jnp.zeros((_RPT, _W1), jnp.float32)
    z_plain = jnp.zeros((_RPT, _D), jnp.float32)

    P1 = _sc_segsum_aug(xaug, src_p, dst_p, z_aug)
    h, rdeg = _tc_layer1(P1, x, W1_l, W1_r, b1.reshape(1, _D))
    P2 = _sc_segsum_plain(h, src_p, dst_p, z_plain)
    return _tc_layer2(P2, h, rdeg, W2_l, W2_r, b2.reshape(1, _D))


# speedup vs baseline: 7.5030x; 5.1414x over previous
import functools

import jax
import jax.numpy as jnp
from jax import lax
from jax.experimental import pallas as pl
from jax.experimental.pallas import tpu as pltpu
from jax.experimental.pallas import tpu_sc as plsc

_N = 10000
_E = 320000
_D = 128
_W1 = 144

_NC = 2
_NS = 16
_NW = _NC * _NS
_CH = 128
_CPW = 79
_EPAD = _NW * _CPW * _CH
_RPT = 640
_NACC = _NS * _RPT

_BN = 1000


def _make_sc_segsum(width):
    mesh = plsc.VectorSubcoreMesh(core_axis_name="c", subcore_axis_name="s")

    @functools.partial(
        pl.kernel,
        out_type=jax.ShapeDtypeStruct((_NC, _NACC, width), jnp.float32),
        mesh=mesh,
        scratch_types=[
            pltpu.VMEM((_CPW, _CH), jnp.int32),
            pltpu.VMEM((_CPW, _CH), jnp.int32),
            pltpu.VMEM((_CH, width), jnp.float32),
            pltpu.VMEM_SHARED((_NACC, width), jnp.float32),
            pltpu.SemaphoreType.DMA,
        ],
        compiler_params=pltpu.CompilerParams(use_tc_tiling_on_sc=False),
    )
    def sc_kernel(table, srcs, dsts, zeros, out,
                  src_v, dst_v, rows, acc, sem):
        c = lax.axis_index("c")
        s = lax.axis_index("s")
        w = s * _NC + c
        pltpu.sync_copy(srcs.at[w], src_v)
        pltpu.sync_copy(dsts.at[w], dst_v)
        pltpu.sync_copy(zeros, acc.at[pl.ds(s * _RPT, _RPT)])
        plsc.subcore_barrier()

        @functools.partial(plsc.parallel_loop, 0, _CPW)
        def step(g):
            pltpu.async_copy(table.at[src_v.at[g]], rows, sem).wait()
            pltpu.sync_copy(rows, acc.at[dst_v.at[g]], add=True)

        plsc.subcore_barrier()
        pltpu.sync_copy(acc.at[pl.ds(s * _RPT, _RPT)],
                        out.at[c, pl.ds(s * _RPT, _RPT)])

    return sc_kernel


_sc_segsum_aug = _make_sc_segsum(_W1)
_sc_segsum_plain = _make_sc_segsum(_D)


def _tc_layer1(P, x, W_l, W_r, b):
    def body(p_ref, x_ref, wl_ref, wr_ref, b_ref, h_ref, rdeg_ref):
        p = p_ref[0] + p_ref[1]
        agg = p[:, :_D]
        deg = p[:, _D:_D + 1]
        rdeg = 1.0 / jnp.maximum(deg, 1.0)
        hv = (jnp.dot(agg * rdeg, wl_ref[:], preferred_element_type=jnp.float32)
              + jnp.dot(x_ref[:], wr_ref[:], preferred_element_type=jnp.float32)
              + b_ref[:])
        h_ref[:] = jnp.maximum(hv, 0.0)
        rdeg_ref[:] = rdeg

    return pl.pallas_call(
        body,
        grid=(_N // _BN,),
        in_specs=[
            pl.BlockSpec((_NC, _BN, _W1), lambda i: (0, i, 0)),
            pl.BlockSpec((_BN, _D), lambda i: (i, 0)),
            pl.BlockSpec((_D, _D), lambda i: (0, 0)),
            pl.BlockSpec((_D, _D), lambda i: (0, 0)),
            pl.BlockSpec((1, _D), lambda i: (0, 0)),
        ],
        out_specs=[
            pl.BlockSpec((_BN, _D), lambda i: (i, 0)),
            pl.BlockSpec((_BN, 1), lambda i: (i, 0)),
        ],
        out_shape=[
            jax.ShapeDtypeStruct((_N, _D), jnp.float32),
            jax.ShapeDtypeStruct((_N, 1), jnp.float32),
        ],
    )(P, x, W_l, W_r, b)


def _tc_layer2(P, h, rdeg, W_l, W_r, b):
    def body(p_ref, h_ref, rdeg_ref, wl_ref, wr_ref, b_ref, o_ref):
        agg = p_ref[0] + p_ref[1]
        o_ref[:] = (jnp.dot(agg * rdeg_ref[:], wl_ref[:],
                            preferred_element_type=jnp.float32)
                    + jnp.dot(h_ref[:], wr_ref[:],
                              preferred_element_type=jnp.float32)
                    + b_ref[:])

    return pl.pallas_call(
        body,
        grid=(_N // _BN,),
        in_specs=[
            pl.BlockSpec((_NC, _BN, _D), lambda i: (0, i, 0)),
            pl.BlockSpec((_BN, _D), lambda i: (i, 0)),
            pl.BlockSpec((_BN, 1), lambda i: (i, 0)),
            pl.BlockSpec((_D, _D), lambda i: (0, 0)),
            pl.BlockSpec((_D, _D), lambda i: (0, 0)),
            pl.BlockSpec((1, _D), lambda i: (0, 0)),
        ],
        out_specs=pl.BlockSpec((_BN, _D), lambda i: (i, 0)),
        out_shape=jax.ShapeDtypeStruct((_N, _D), jnp.float32),
    )(P, h, rdeg, W_l, W_r, b)


def kernel(x, edge_index, W1_l, W1_r, b1, W2_l, W2_r, b2):
    src = edge_index[0]
    dst = edge_index[1]
    pad = _EPAD - _E
    src_p = jnp.concatenate(
        [src, jnp.zeros((pad,), jnp.int32)]).reshape(
            _CPW, _NW, _CH).transpose(1, 0, 2)
    pad_dst = _N + (jnp.arange(pad, dtype=jnp.int32) % (_NACC - _N))
    dst_p = jnp.concatenate([dst, pad_dst]).reshape(
        _CPW, _NW, _CH).transpose(1, 0, 2)
    xaug = jnp.concatenate(
        [x, jnp.ones((_N, 1), jnp.float32), jnp.zeros((_N, _W1 - _D - 1),
                                                      jnp.float32)], axis=1)
    z_aug = jnp.zeros((_RPT, _W1), jnp.float32)
    z_plain = jnp.zeros((_RPT, _D), jnp.float32)

    P1 = _sc_segsum_aug(xaug, src_p, dst_p, z_aug)
    h, rdeg = _tc_layer1(P1, x, W1_l, W1_r, b1.reshape(1, _D))
    P2 = _sc_segsum_plain(h, src_p, dst_p, z_plain)
    return _tc_layer2(P2, h, rdeg, W2_l, W2_r, b2.reshape(1, _D))
